# Initial kernel scaffold; baseline (speedup 1.0000x reference)
import jax
import jax.numpy as jnp
from jax.experimental import pallas as pl


def kernel(z, edge_index, edge_attr, batch, node_emb, Wf0, bf0, Ws0, bs0, gamma0, beta0, Wf1, bf1, Ws1, bs1, gamma1, beta1, Wf2, bf2, Ws2, bs2, gamma2, beta2, W1, b1, W2, b2):
    raise NotImplementedError("write your pallas kernel here")


# speedup vs baseline: 1.9065x; 1.9065x over previous
import functools

import jax
import jax.numpy as jnp
from jax import lax
from jax.experimental import pallas as pl
from jax.experimental.pallas import tpu as pltpu
from jax.experimental.pallas import tpu_sc as plsc

N = 50000
E = 800000
EMB = 64
EDGE = 50
EDGE_P = 64
HID = 128
NG = 64
MAXZ = 118


BN_N = 2000
NB_N = N // BN_N
BN_E = 4000
NB_E = E // BN_E

_F32 = jnp.float32


def _embed_body(z_ref, emb_ref, wd_ref, wsc_ref, x_ref, a_ref, b_ref):
    z = jnp.clip(z_ref[...], 0, MAXZ)
    iot = lax.broadcasted_iota(jnp.int32, (BN_N, 128), 1)
    oh = (z == iot).astype(_F32)
    x = jnp.dot(oh, emb_ref[...], preferred_element_type=_F32)
    x_ref[...] = x
    a_ref[...] = jnp.dot(x, wd_ref[...], preferred_element_type=_F32)
    b_ref[...] = jnp.dot(x, wsc_ref[...], preferred_element_type=_F32)


def _k_embed(z2, emb_p, wd, wsc):
    return pl.pallas_call(
        _embed_body,
        grid=(NB_N,),
        in_specs=[
            pl.BlockSpec((BN_N, 1), lambda i: (i, 0)),
            pl.BlockSpec((128, EMB), lambda i: (0, 0)),
            pl.BlockSpec((EMB, 128), lambda i: (0, 0)),
            pl.BlockSpec((EMB, 128), lambda i: (0, 0)),
        ],
        out_specs=[
            pl.BlockSpec((BN_N, EMB), lambda i: (i, 0)),
            pl.BlockSpec((BN_N, 128), lambda i: (i, 0)),
            pl.BlockSpec((BN_N, 128), lambda i: (i, 0)),
        ],
        out_shape=[
            jax.ShapeDtypeStruct((N, EMB), _F32),
            jax.ShapeDtypeStruct((N, 128), _F32),
            jax.ShapeDtypeStruct((N, 128), _F32),
        ],
    )(z2, emb_p, wd, wsc)


def _act_body(g_ref, ea_ref, we_ref, bias_ref, msg_ref):
    u = (g_ref[...]
         + jnp.dot(ea_ref[...], we_ref[...], preferred_element_type=_F32)
         + bias_ref[...])
    uf = u[:, :EMB]
    us = u[:, EMB:]
    msg_ref[...] = jax.nn.sigmoid(uf) * jax.nn.softplus(us)


def _k_act(g, ea_p, we_p, bias):
    return pl.pallas_call(
        _act_body,
        grid=(NB_E,),
        in_specs=[
            pl.BlockSpec((BN_E, 128), lambda i: (i, 0)),
            pl.BlockSpec((BN_E, EDGE_P), lambda i: (i, 0)),
            pl.BlockSpec((EDGE_P, 128), lambda i: (0, 0)),
            pl.BlockSpec((1, 128), lambda i: (0, 0)),
        ],
        out_specs=pl.BlockSpec((BN_E, EMB), lambda i: (i, 0)),
        out_shape=jax.ShapeDtypeStruct((E, EMB), _F32),
    )(g, ea_p, we_p, bias)


def _stats_body(agg_ref, o_ref):
    @pl.when(pl.program_id(0) == 0)
    def _():
        o_ref[...] = jnp.zeros_like(o_ref)

    a = agg_ref[...]
    s = jnp.sum(a, axis=0, keepdims=True)
    sq = jnp.sum(a * a, axis=0, keepdims=True)
    o_ref[...] += jnp.concatenate([s, sq], axis=0)


def _k_stats(agg):
    return pl.pallas_call(
        _stats_body,
        grid=(NB_N,),
        in_specs=[pl.BlockSpec((BN_N, EMB), lambda i: (i, 0))],
        out_specs=pl.BlockSpec((2, EMB), lambda i: (0, 0)),
        out_shape=jax.ShapeDtypeStruct((2, EMB), _F32),
    )(agg)


def _bn_common(agg_ref, x_ref, st_ref, gb_ref):
    st = st_ref[...]
    mu = st[0:1, :] * (1.0 / N)
    var = st[1:2, :] * (1.0 / N) - mu * mu
    inv = lax.rsqrt(var + 1e-5)
    bn = (agg_ref[...] - mu) * inv * gb_ref[0:1, :] + gb_ref[1:2, :]
    t = bn + x_ref[...]
    return t * jax.nn.sigmoid(t)


def _bn_body(agg_ref, x_ref, st_ref, gb_ref, wd_ref, wsc_ref,
             xo_ref, a_ref, b_ref):
    xn = _bn_common(agg_ref, x_ref, st_ref, gb_ref)
    xo_ref[...] = xn
    a_ref[...] = jnp.dot(xn, wd_ref[...], preferred_element_type=_F32)
    b_ref[...] = jnp.dot(xn, wsc_ref[...], preferred_element_type=_F32)


def _bn_last_body(agg_ref, x_ref, st_ref, gb_ref, xo_ref):
    xo_ref[...] = _bn_common(agg_ref, x_ref, st_ref, gb_ref)


def _k_bn(agg, x, st, gb, wd, wsc):
    return pl.pallas_call(
        _bn_body,
        grid=(NB_N,),
        in_specs=[
            pl.BlockSpec((BN_N, EMB), lambda i: (i, 0)),
            pl.BlockSpec((BN_N, EMB), lambda i: (i, 0)),
            pl.BlockSpec((2, EMB), lambda i: (0, 0)),
            pl.BlockSpec((2, EMB), lambda i: (0, 0)),
            pl.BlockSpec((EMB, 128), lambda i: (0, 0)),
            pl.BlockSpec((EMB, 128), lambda i: (0, 0)),
        ],
        out_specs=[
            pl.BlockSpec((BN_N, EMB), lambda i: (i, 0)),
            pl.BlockSpec((BN_N, 128), lambda i: (i, 0)),
            pl.BlockSpec((BN_N, 128), lambda i: (i, 0)),
        ],
        out_shape=[
            jax.ShapeDtypeStruct((N, EMB), _F32),
            jax.ShapeDtypeStruct((N, 128), _F32),
            jax.ShapeDtypeStruct((N, 128), _F32),
        ],
    )(agg, x, st, gb, wd, wsc)


def _k_bn_last(agg, x, st, gb):
    return pl.pallas_call(
        _bn_last_body,
        grid=(NB_N,),
        in_specs=[
            pl.BlockSpec((BN_N, EMB), lambda i: (i, 0)),
            pl.BlockSpec((BN_N, EMB), lambda i: (i, 0)),
            pl.BlockSpec((2, EMB), lambda i: (0, 0)),
            pl.BlockSpec((2, EMB), lambda i: (0, 0)),
        ],
        out_specs=pl.BlockSpec((BN_N, EMB), lambda i: (i, 0)),
        out_shape=jax.ShapeDtypeStruct((N, EMB), _F32),
    )(agg, x, st, gb)


def _pool_body(x_ref, bt_ref, w1_ref, b1_ref, w2_ref, b2_ref, y_ref,
               acc_ref, cnt_ref):
    i = pl.program_id(0)

    @pl.when(i == 0)
    def _():
        acc_ref[...] = jnp.zeros_like(acc_ref)
        cnt_ref[...] = jnp.zeros_like(cnt_ref)

    bt = bt_ref[...]
    oh = (bt == lax.broadcasted_iota(jnp.int32, (BN_N, NG), 1)).astype(_F32)
    acc_ref[...] += lax.dot_general(oh, x_ref[...], (((0,), (0,)), ((), ())),
                                    preferred_element_type=_F32)
    cnt_ref[...] += lax.dot_general(oh, jnp.ones((BN_N, 1), _F32),
                                    (((0,), (0,)), ((), ())),
                                    preferred_element_type=_F32)

    @pl.when(i == NB_N - 1)
    def _():
        gp = acc_ref[...] / jnp.maximum(cnt_ref[...], 1.0)
        h = jnp.dot(gp, w1_ref[...], preferred_element_type=_F32) + b1_ref[...]
        h = h * jax.nn.sigmoid(h)
        y_ref[...] = jnp.dot(h, w2_ref[...], preferred_element_type=_F32) + b2_ref[...]


def _k_pool(x, bt2, w1, b1r, w2, b2r):
    return pl.pallas_call(
        _pool_body,
        grid=(NB_N,),
        in_specs=[
            pl.BlockSpec((BN_N, EMB), lambda i: (i, 0)),
            pl.BlockSpec((BN_N, 1), lambda i: (i, 0)),
            pl.BlockSpec((EMB, HID), lambda i: (0, 0)),
            pl.BlockSpec((1, HID), lambda i: (0, 0)),
            pl.BlockSpec((HID, 1), lambda i: (0, 0)),
            pl.BlockSpec((1, 1), lambda i: (0, 0)),
        ],
        out_specs=pl.BlockSpec((NG, 1), lambda i: (0, 0)),
        out_shape=jax.ShapeDtypeStruct((NG, 1), _F32),
        scratch_shapes=[
            pltpu.VMEM((NG, EMB), _F32),
            pltpu.VMEM((NG, 1), _F32),
        ],
    )(x, bt2, w1, b1r, w2, b2r)


NW = 32
EW = E // NW
GCH = 128
G_FULL = EW // GCH
G_TAIL = EW - G_FULL * GCH

NH = N // 2
SLAB = 25088
ET = E // 16
S_FULL = ET // GCH
S_TAIL = ET - S_FULL * GCH
ZROWS = SLAB // 16
ZCH = 112
WROWS = 1560
WREM = NH - 16 * WROWS

@functools.cache
def _sc_mesh():
    return plsc.VectorSubcoreMesh(core_axis_name="c", subcore_axis_name="s")


def _gather_sc(a_hbm, b_hbm, dst_hbm, src_hbm, g_hbm,
               idxd, idxs, bufa, bufb, idxdt, idxst, bufat, bufbt,
               sema, semb):
    c = lax.axis_index("c")
    s = lax.axis_index("s")
    wid = s * 2 + c
    base0 = wid * EW

    def do_chunk(base, cl, ixd, ixs, ba, bb):
        pltpu.sync_copy(dst_hbm.at[pl.ds(base, cl)], ixd)
        pltpu.sync_copy(src_hbm.at[pl.ds(base, cl)], ixs)
        cpa = pltpu.async_copy(a_hbm.at[ixd], ba, sema)
        cpb = pltpu.async_copy(b_hbm.at[ixs], bb, semb)
        cpa.wait()
        cpb.wait()

        def addrow(r, carry):
            for u in range(8):
                sl = pl.ds(u * 16, 16)
                ba[r, sl] = ba[r, sl] + bb[r, sl]
            return carry

        lax.fori_loop(0, cl, addrow, 0)
        pltpu.sync_copy(ba, g_hbm.at[pl.ds(base, cl), :])

    def loop_body(ci, carry):
        do_chunk(base0 + ci * GCH, GCH, idxd, idxs, bufa, bufb)
        return carry

    lax.fori_loop(0, G_FULL, loop_body, 0)
    do_chunk(base0 + G_FULL * GCH, G_TAIL, idxdt, idxst, bufat, bufbt)


def _k_gather(a, b, dst, src):
    f = pl.kernel(
        _gather_sc,
        out_type=jax.ShapeDtypeStruct((E, 128), _F32),
        mesh=_sc_mesh(),
        scratch_types=[
            pltpu.VMEM((GCH,), jnp.int32),
            pltpu.VMEM((GCH,), jnp.int32),
            pltpu.VMEM((GCH, 128), _F32),
            pltpu.VMEM((GCH, 128), _F32),
            pltpu.VMEM((G_TAIL,), jnp.int32),
            pltpu.VMEM((G_TAIL,), jnp.int32),
            pltpu.VMEM((G_TAIL, 128), _F32),
            pltpu.VMEM((G_TAIL, 128), _F32),
            pltpu.SemaphoreType.DMA,
            pltpu.SemaphoreType.DMA,
        ],
    )
    return f(a, b, dst, src)


def _scatter_sc(msg_hbm, dst_hbm, agg_hbm, slab, idxb, msgb, idxbt, ramp, widx):
    c = lax.axis_index("c")
    s = lax.axis_index("s")
    nbase = c * NH
    i16 = lax.iota(jnp.int32, 16)
    zero16 = jnp.zeros((16,), _F32)

    def zrow(r, carry):
        for u in range(4):
            msgb[r, pl.ds(u * 16, 16)] = zero16
        return carry

    lax.fori_loop(0, GCH, zrow, 0)

    def rrow(r, carry):
        ramp[0, pl.ds(r * 16, 16)] = r * 16 + i16
        return carry

    lax.fori_loop(0, GCH // 16, rrow, 0)

    zb = s * ZROWS

    def zcp(k, carry):
        off = zb + k * ZCH

        def ib(r, carry2):
            sl = pl.ds(r * 16, 16)
            widx[0, sl] = ramp[0, sl] + off
            return carry2

        lax.fori_loop(0, ZCH // 16, ib, 0)
        pltpu.sync_copy(msgb.at[pl.ds(0, ZCH), :], slab.at[widx.at[0]])
        return carry

    lax.fori_loop(0, ZROWS // ZCH, zcp, 0)
    plsc.subcore_barrier()

    ebase = s * ET

    def do_chunk(base, cl, ixr, mr):
        pltpu.sync_copy(dst_hbm.at[pl.ds(base, cl)], ixr.at[0])
        for u in range(cl // 16):
            sl = pl.ds(u * 16, 16)
            v = ixr[0, sl]
            lv = v - nbase
            oob = (lv < 0) | (lv >= NH)
            ixr[0, sl] = jnp.where(oob, NH, lv)
        pltpu.sync_copy(msg_hbm.at[pl.ds(base, cl), :], mr)
        pltpu.sync_copy(mr, slab.at[ixr.at[0]], add=True)

    def loop_body(ci, carry):
        do_chunk(ebase + ci * GCH, GCH, idxb, msgb)
        return carry

    lax.fori_loop(0, S_FULL, loop_body, 0)
    do_chunk(ebase + S_FULL * GCH, S_TAIL, idxbt, msgb.at[pl.ds(0, S_TAIL), :])
    plsc.subcore_barrier()

    wb = s * WROWS

    def mkidx(off):
        def ib(r, carry2):
            sl = pl.ds(r * 16, 16)
            idxb[0, sl] = ramp[0, sl] + off
            return carry2

        lax.fori_loop(0, GCH // 16, ib, 0)

    def wcp(k, carry):
        off = wb + k * GCH
        mkidx(off)
        pltpu.sync_copy(slab.at[idxb.at[0]], msgb)
        pltpu.sync_copy(msgb, agg_hbm.at[pl.ds(nbase + off, GCH), :])
        return carry

    lax.fori_loop(0, WROWS // GCH, wcp, 0)
    wtail = WROWS % GCH
    off_t = wb + (WROWS // GCH) * GCH
    mkidx(off_t)
    pltpu.sync_copy(slab.at[idxb.at[0, pl.ds(0, wtail)]],
                    msgb.at[pl.ds(0, wtail), :])
    pltpu.sync_copy(msgb.at[pl.ds(0, wtail), :],
                    agg_hbm.at[pl.ds(nbase + off_t, wtail), :])

    @pl.when(s == 15)
    def _():
        off_r = 16 * WROWS
        mkidx(off_r)
        pltpu.sync_copy(slab.at[idxb.at[0, pl.ds(0, WREM)]],
                        msgb.at[pl.ds(0, WREM), :])
        pltpu.sync_copy(msgb.at[pl.ds(0, WREM), :],
                        agg_hbm.at[pl.ds(nbase + off_r, WREM), :])


def _k_scatter(msg, dst):
    f = pl.kernel(
        _scatter_sc,
        out_type=jax.ShapeDtypeStruct((N, EMB), _F32),
        mesh=_sc_mesh(),
        scratch_types=[
            pltpu.VMEM_SHARED((SLAB, EMB), _F32),
            pltpu.VMEM((1, GCH), jnp.int32),
            pltpu.VMEM((GCH, EMB), _F32),
            pltpu.VMEM((1, S_TAIL), jnp.int32),
            pltpu.VMEM((1, GCH), jnp.int32),
            pltpu.VMEM((1, ZCH), jnp.int32),
        ],
    )
    return f(msg, dst)


def kernel(z, edge_index, edge_attr, batch, node_emb,
           Wf0, bf0, Ws0, bs0, gamma0, beta0,
           Wf1, bf1, Ws1, bs1, gamma1, beta1,
           Wf2, bf2, Ws2, bs2, gamma2, beta2,
           W1, b1, W2, b2):
    z2 = z.astype(jnp.int32).reshape(N, 1)
    dst = edge_index[1].astype(jnp.int32)
    src = edge_index[0].astype(jnp.int32)
    bt2 = batch.astype(jnp.int32).reshape(N, 1)
    ea_p = jnp.pad(edge_attr, ((0, 0), (0, EDGE_P - EDGE)))
    emb_p = jnp.pad(node_emb, ((0, 128 - (MAXZ + 1)), (0, 0)))

    layers = [
        (Wf0, bf0, Ws0, bs0, gamma0, beta0),
        (Wf1, bf1, Ws1, bs1, gamma1, beta1),
        (Wf2, bf2, Ws2, bs2, gamma2, beta2),
    ]
    wds, wscs, weps, biases, gbs = [], [], [], [], []
    for (Wf, bf, Ws, bs, gm, bb) in layers:
        wds.append(jnp.concatenate([Wf[:EMB], Ws[:EMB]], axis=1))
        wscs.append(jnp.concatenate([Wf[EMB:2 * EMB], Ws[EMB:2 * EMB]], axis=1))
        weps.append(jnp.pad(jnp.concatenate([Wf[2 * EMB:], Ws[2 * EMB:]], axis=1),
                            ((0, EDGE_P - EDGE), (0, 0))))
        biases.append(jnp.concatenate([bf, bs]).reshape(1, 128))
        gbs.append(jnp.stack([gm, bb]))

    x, a, b = _k_embed(z2, emb_p, wds[0], wscs[0])
    for i in range(3):
        g = _k_gather(a, b, dst, src)
        msg = _k_act(g, ea_p, weps[i], biases[i])
        agg = jnp.zeros((N, EMB), _F32).at[dst].add(msg)
        st = _k_stats(agg)
        if i < 2:
            x, a, b = _k_bn(agg, x, st, gbs[i], wds[i + 1], wscs[i + 1])
        else:
            x = _k_bn_last(agg, x, st, gbs[i])

    y = _k_pool(x, bt2, W1, b1.reshape(1, HID), W2, b2.reshape(1, 1))
    return y.reshape(NG)
